# Initial kernel scaffold; baseline (speedup 1.0000x reference)
import jax
import jax.numpy as jnp
from jax.experimental import pallas as pl


def kernel(roi_boxes3d, gt_boxes3d, rpn_xyz, pts_feature):
    raise NotImplementedError("write your pallas kernel here")


# speedup vs baseline: 6.5710x; 6.5710x over previous
import functools
import numpy as np
import jax
import jax.numpy as jnp
from jax import lax
from jax.experimental import pallas as pl
from jax.experimental.pallas import tpu as pltpu
from jax.experimental.pallas import tpu_sc as plsc

_B, _M, _NGT, _NPTS, _C = 4, 512, 64, 16384, 128
_NROI = 64
_FG = 32
_NP = 512
_NSEL = _B * _NROI
_NW = 32
_RPW = _NSEL // _NW
_LIST = _NP + 16


def _select_body(roi_ref, gt_ref, out_ref):
    r = roi_ref[0]
    g = gt_ref[0]
    rx, ry_, rz = r[0], r[1], r[2]
    rh, rw, rl = r[3], r[4], r[5]
    gx, gy, gz = g[0, :_NGT], g[1, :_NGT], g[2, :_NGT]
    gh, gw, gl = g[3, :_NGT], g[4, :_NGT], g[5, :_NGT]
    gry = g[6, :_NGT]

    def inter1d(c1, s1, c2, s2):
        lo = jnp.maximum(c1 - s1 * 0.5, c2 - s2 * 0.5)
        hi = jnp.minimum(c1 + s1 * 0.5, c2 + s2 * 0.5)
        return jnp.clip(hi - lo, 0.0, None)

    ix = inter1d(rx[:, None], rl[:, None], gx[None, :], gl[None, :])
    iy = inter1d(ry_[:, None], rh[:, None], gy[None, :], gh[None, :])
    iz = inter1d(rz[:, None], rw[:, None], gz[None, :], gw[None, :])
    inter = ix * iy * iz
    vr = (rh * rw * rl)[:, None]
    vg = (gh * gw * gl)[None, :]
    iou = inter / jnp.maximum(vr + vg - inter, 1e-6)

    mo = jnp.max(iou, axis=1)
    gcol = lax.broadcasted_iota(jnp.int32, (_M, _NGT), 1)
    ga = jnp.min(jnp.where(iou == mo[:, None], gcol, _NGT), axis=1)

    moi = mo[:, None]
    ii = lax.broadcasted_iota(jnp.int32, (_M, _NGT), 0)
    jj0 = lax.broadcasted_iota(jnp.int32, (_M, _NGT), 1)
    rk = jnp.zeros((_M,), jnp.int32)
    sk = jnp.zeros((_M,), jnp.int32)
    for jb in range(_M // _NGT):
        moj = mo[jb * _NGT:(jb + 1) * _NGT][None, :]
        eqlt = (moj == moi) & ((jj0 + jb * _NGT) < ii)
        rk = rk + jnp.sum(((moj > moi) | eqlt).astype(jnp.int32), axis=1)
        sk = sk + jnp.sum(((moj < moi) | eqlt).astype(jnp.int32), axis=1)

    kk = lax.broadcasted_iota(jnp.int32, (_NROI, _M), 0)
    oh = (((rk[None, :] == kk) & (kk < _FG)) |
          ((sk[None, :] == (kk - _FG)) & (kk >= _FG))).astype(jnp.float32)

    def sel(plane):
        return jnp.sum(oh * plane[None, :], axis=1)

    scx, scy, scz = sel(rx), sel(ry_), sel(rz)
    sh, sw, sl = sel(rh), sel(rw), sel(rl)
    sry = sel(r[6])
    siou = sel(mo)
    sga = sel(ga.astype(jnp.float32)).astype(jnp.int32)

    oh2 = (sga[:, None] == lax.broadcasted_iota(jnp.int32, (_NROI, _NGT), 1)
           ).astype(jnp.float32)

    def sel2(plane):
        return jnp.sum(oh2 * plane[None, :], axis=1)

    gsx, gsy, gsz = sel2(gx), sel2(gy), sel2(gz)
    gsh, gsw, gsl = sel2(gh), sel2(gw), sel2(gl)
    gsry = sel2(gry)

    rym = jnp.mod(sry, 2.0 * np.pi)
    cosa = jnp.cos(rym)
    sina = jnp.sin(rym)
    dx, dy, dz = gsx - scx, gsy - scy, gsz - scz
    gnx = cosa * dx - sina * dz
    gnz = sina * dx + cosa * dz

    cls = (siou > 0.6).astype(jnp.float32)
    invalid = (siou > 0.45) & (siou < 0.6)
    cls = jnp.where(invalid, -1.0, cls)
    rvm = (siou > 0.55).astype(jnp.float32)

    zeros = jnp.zeros((_NROI,), jnp.float32)
    planes = [
        scx, scy, scz,
        (sl + 2.0) * 0.5, (sh + 2.0) * 0.5, (sw + 2.0) * 0.5,
        cosa, sina,
        scx, scy, scz, sh, sw, sl, sry, zeros,
        gnx, dy, gnz, gsh, gsw, gsl, gsry - rym, zeros,
        siou, cls, rvm, zeros, zeros, zeros, zeros, zeros,
    ]
    stacked = jnp.stack(planes, axis=0)
    out_ref[0] = jnp.concatenate(
        [stacked, jnp.zeros((32, _M // 4 - _NROI), jnp.float32)], axis=1)


def _run_select(roi_planes, gt_planes):
    return pl.pallas_call(
        _select_body,
        grid=(_B,),
        in_specs=[
            pl.BlockSpec((1, 8, _M), lambda b: (b, 0, 0)),
            pl.BlockSpec((1, 8, 128), lambda b: (b, 0, 0)),
        ],
        out_specs=pl.BlockSpec((1, 32, 128), lambda b: (b, 0, 0)),
        out_shape=jax.ShapeDtypeStruct((_B, 32, 128), jnp.float32),
    )(roi_planes, gt_planes)


def _sc_sample(xyz_flat, feat_flat, params):
    mesh = plsc.VectorSubcoreMesh(core_axis_name="c", subcore_axis_name="s")
    info = plsc.get_sparse_core_info()
    nc = info.num_cores

    @functools.partial(
        pl.kernel, mesh=mesh,
        compiler_params=pltpu.CompilerParams(needs_layout_passes=False),
        out_type=[
            jax.ShapeDtypeStruct((_NSEL, _NP * 3), jnp.float32),
            jax.ShapeDtypeStruct((_NSEL, _NP, _C), jnp.float32),
            jax.ShapeDtypeStruct((_NW, 16), jnp.int32),
        ],
        scratch_types=[
            pltpu.VMEM((_NPTS * 3,), jnp.float32),
            pltpu.VMEM((_RPW * 16,), jnp.float32),
            pltpu.VMEM((_LIST,), jnp.int32),
            pltpu.VMEM((4, 128), jnp.int32),
            pltpu.VMEM((_NP * 3,), jnp.float32),
            pltpu.VMEM((_NP, _C), jnp.float32),
            pltpu.VMEM((16,), jnp.int32),
            pltpu.SemaphoreType.DMA,
        ],
    )
    def body(xyz_hbm, feat_hbm, par_hbm, pts_out, feat_out, emp_out,
             xyz_v, par_v, idx_v, gidx_v, pts_v, feat_v, emp_v, sem):
        wid = lax.axis_index("s") * nc + lax.axis_index("c")
        base = wid * _RPW
        batch = base // _NROI
        pltpu.sync_copy(xyz_hbm.at[pl.ds(batch * (_NPTS * 3), _NPTS * 3)], xyz_v)
        pltpu.sync_copy(par_hbm.at[pl.ds(base * 16, _RPW * 16)], par_v)
        emp_v[...] = jnp.zeros((16,), jnp.int32)
        lanes = lax.iota(jnp.int32, 16)

        def per_roi(ri, carry):
            def par(j):
                return plsc.load_gather(
                    par_v, [jnp.full((16,), ri * 16 + j, jnp.int32)])

            cx, cy, cz = par(0), par(1), par(2)
            ex2, ey2, ez2 = par(3), par(4), par(5)
            cosa, sina = par(6), par(7)

            def chunk_mask(c):
                pidx = c * 16 + lanes
                xi = plsc.load_gather(xyz_v, [pidx * 3])
                yi = plsc.load_gather(xyz_v, [pidx * 3 + 1])
                zi = plsc.load_gather(xyz_v, [pidx * 3 + 2])
                m = ((jnp.abs(xi - cx) <= ex2) &
                     (jnp.abs(yi - cy) <= ey2) &
                     (jnp.abs(zi - cz) <= ez2))
                return pidx, m

            def p1_cond(st):
                c, cnt = st
                return (c < _NPTS // 16) & (cnt < _NP)

            def p1_body(st):
                c, cnt = st
                pidx, m = chunk_mask(c)
                plsc.store_compressed(idx_v.at[pl.ds(cnt, 16)], pidx, mask=m)
                n = jnp.max(plsc.all_reduce_population_count(m))
                return c + 1, cnt + n

            _, n_in = lax.while_loop(p1_cond, p1_body, (0, 0))

            def p2_cond(st):
                c, tot = st
                return (c < _NPTS // 16) & (tot < _NP)

            def p2_body(st):
                c, tot = st
                pidx, m = chunk_mask(c)
                mo_ = ~m
                plsc.store_compressed(idx_v.at[pl.ds(tot, 16)], pidx, mask=mo_)
                n = jnp.max(plsc.all_reduce_population_count(mo_))
                return c + 1, tot + n

            lax.while_loop(p2_cond, p2_body, (0, n_in))
            emp_v[...] = jnp.where(lanes == ri, (n_in == 0).astype(jnp.int32),
                                   emp_v[...])

            def xform(k, _):
                li = idx_v[pl.ds(k * 16, 16)]
                xi = plsc.load_gather(xyz_v, [li * 3])
                yi = plsc.load_gather(xyz_v, [li * 3 + 1])
                zi = plsc.load_gather(xyz_v, [li * 3 + 2])
                dx = xi - cx
                dz = zi - cz
                lo = k * 16 + lanes
                plsc.store_scatter(pts_v, [lo * 3], cosa * dx - sina * dz)
                plsc.store_scatter(pts_v, [lo * 3 + 1], yi - cy)
                plsc.store_scatter(pts_v, [lo * 3 + 2], sina * dx + cosa * dz)
                plsc.store_scatter(gidx_v, [lo // 128, lo % 128],
                                   li + batch * _NPTS)
                return 0

            lax.fori_loop(0, _NP // 16, xform, 0)

            for j in range(4):
                pltpu.async_copy(feat_hbm.at[gidx_v.at[j]],
                                 feat_v.at[pl.ds(j * 128, 128)], sem).wait()

            pltpu.sync_copy(pts_v, pts_out.at[base + ri])
            pltpu.sync_copy(feat_v, feat_out.at[base + ri])
            return carry

        lax.fori_loop(0, _RPW, per_roi, 0)
        pltpu.sync_copy(emp_v, emp_out.at[wid])

    return body(xyz_flat, feat_flat, params)


def kernel(roi_boxes3d, gt_boxes3d, rpn_xyz, pts_feature):
    roi_planes = jnp.concatenate(
        [jnp.transpose(roi_boxes3d, (0, 2, 1)),
         jnp.zeros((_B, 1, _M), jnp.float32)], axis=1)
    gt_planes = jnp.concatenate(
        [jnp.transpose(gt_boxes3d, (0, 2, 1)),
         jnp.zeros((_B, 8, 128 - _NGT), jnp.float32)], axis=2)

    out = _run_select(roi_planes, gt_planes)
    v = out[:, :, :_NROI]
    params = jnp.transpose(v[:, 0:8], (0, 2, 1)).reshape(_NSEL, 8)
    rois = jnp.transpose(v[:, 8:15], (0, 2, 1)).reshape(_NSEL, 7)
    gtc = jnp.transpose(v[:, 16:23], (0, 2, 1)).reshape(_NSEL, 7)
    giou = v[:, 24].reshape(_NSEL)
    cls_pre = v[:, 25].reshape(_NSEL).astype(jnp.int32)
    rvm_pre = v[:, 26].reshape(_NSEL).astype(jnp.int32)

    params16 = jnp.concatenate(
        [params, jnp.zeros((_NSEL, 8), jnp.float32)], axis=1).reshape(-1)
    sp_flat, sf, emp2d = _sc_sample(
        rpn_xyz.reshape(-1), pts_feature.reshape(_B * _NPTS, _C), params16)
    sp = sp_flat.reshape(_NSEL, _NP, 3)
    empty = emp2d[:, :_RPW].reshape(_NSEL)

    cls_label = jnp.where(empty == 1, -1, cls_pre)
    rvm = rvm_pre * (1 - empty)
    return sp, sf, cls_label, rvm, gtc, giou, rois
